# baseline (speedup 1.0000x reference)
import dataclasses
import functools

import jax
import jax.numpy as jnp
from jax import lax
from jax.experimental import pallas as pl
from jax.experimental.pallas import tpu as pltpu
from jax.experimental.pallas import tpu_sc as plsc

N = 10000
E = 320000
D = 128
NC = 2
NS = 16
NW = NC * NS
EPW = E // NW
CH = 80
NCH = EPW // CH
CC = 40
SUPC = 10
NSUP = EPW // (SUPC * CC)
NBUF = 5
NP = 10240
SUBN = NP // NS
L = 16

_mesh = plsc.VectorSubcoreMesh(
    core_axis_name="c", subcore_axis_name="s", num_cores=NC, num_subcores=NS
)

_f32 = jnp.float32

_sc_params = pltpu.CompilerParams()
if "needs_layout_passes" in pltpu.CompilerParams.__dataclass_fields__:
    _sc_params = dataclasses.replace(_sc_params, needs_layout_passes=False)


def _worker_ids():
    cid = lax.axis_index("c")
    sid = lax.axis_index("s")
    return cid, sid, cid * NS + sid


def _zero_vec(vec_ref):
    @pl.loop(0, vec_ref.shape[0] // L)
    def _(t):
        vec_ref[pl.ds(t * L, L)] = jnp.zeros((L,), _f32)


def _zero_shared_table(sh_ref, zero_ref, sid):
    pltpu.sync_copy(zero_ref, sh_ref.at[pl.ds(sid * SUBN, SUBN)])


def _combine_partials(dst_ref, tmp_ref, part_hbm):
    pltpu.sync_copy(part_hbm.at[0], dst_ref)
    pltpu.sync_copy(part_hbm.at[1], tmp_ref)

    @plsc.parallel_loop(0, NP // L, unroll=4)
    def _(t):
        s = pl.ds(t * L, L)
        dst_ref[s] = dst_ref[s] + tmp_ref[s]


def _export_table(sh_ref, part_hbm, cid, sid):
    s = pl.ds(sid * SUBN, SUBN)
    pltpu.sync_copy(sh_ref.at[s], part_hbm.at[cid, s])


@functools.partial(
    pl.kernel,
    out_type=[
        jax.ShapeDtypeStruct((NW, EPW), _f32),
        jax.ShapeDtypeStruct((NC, NP), _f32),
        jax.ShapeDtypeStruct((NC, NP), _f32),
    ],
    mesh=_mesh,
    compiler_params=_sc_params,
    scratch_types=[
        pltpu.VMEM((EPW,), _f32),
        pltpu.VMEM((NCH, CH), jnp.int32),
        pltpu.VMEM((NCH, CH), jnp.int32),
        pltpu.VMEM((SUBN,), _f32),
        pltpu.VMEM_SHARED((NP,), _f32),
        pltpu.VMEM_SHARED((NP,), _f32),
        pltpu.SemaphoreType.DMA,
        pltpu.SemaphoreType.DMA,
    ],
)
def _stage_a(val_hbm, i_hbm, j_hbm, ev_hbm, sip_hbm, sjp_hbm,
             val_v, i_v, j_v, zero_v, si_sh, sj_sh, sem_i, sem_j):
    cid, sid, wid = _worker_ids()
    _zero_vec(zero_v)
    _zero_shared_table(si_sh, zero_v, sid)
    _zero_shared_table(sj_sh, zero_v, sid)
    pltpu.sync_copy(val_hbm.at[wid], val_v)
    pltpu.sync_copy(i_hbm.at[wid], i_v)
    pltpu.sync_copy(j_hbm.at[wid], j_v)

    @plsc.parallel_loop(0, EPW // L, unroll=4)
    def _(t):
        s = pl.ds(t * L, L)
        val_v[s] = jnp.exp(val_v[s])

    plsc.subcore_barrier()

    @pl.loop(0, NCH, step=5)
    def _(b):
        descs = []
        for t in range(5):
            src = val_v.at[pl.ds((b + t) * CH, CH)]
            descs.append(pltpu.async_copy(src, si_sh.at[i_v.at[b + t]],
                                          sem_i, add=True))
            descs.append(pltpu.async_copy(src, sj_sh.at[j_v.at[b + t]],
                                          sem_j, add=True))
        for d in descs:
            d.wait()

    pltpu.sync_copy(val_v, ev_hbm.at[wid])
    plsc.subcore_barrier()
    _export_table(si_sh, sip_hbm, cid, sid)
    _export_table(sj_sh, sjp_hbm, cid, sid)


@functools.partial(
    pl.kernel,
    out_type=[
        jax.ShapeDtypeStruct((NW, EPW), _f32),
        jax.ShapeDtypeStruct((NC, NP), _f32),
    ],
    mesh=_mesh,
    compiler_params=_sc_params,
    scratch_types=[
        pltpu.VMEM((EPW,), _f32),
        pltpu.VMEM((NCH, CH), jnp.int32),
        pltpu.VMEM((NCH, CH), jnp.int32),
        pltpu.VMEM((NP,), _f32),
        pltpu.VMEM((NP,), _f32),
        pltpu.VMEM((NP,), _f32),
        pltpu.VMEM((SUBN,), _f32),
        pltpu.VMEM_SHARED((NP,), _f32),
        pltpu.SemaphoreType.DMA,
    ],
)
def _stage_b(ev_hbm, i_hbm, j_hbm, sip_hbm, sjp_hbm, w_hbm, sep_hbm,
             ev_v, i_v, j_v, si_t, sj_t, tmp_v, zero_v, se_sh, sem_e):
    cid, sid, wid = _worker_ids()
    _zero_vec(zero_v)
    _zero_shared_table(se_sh, zero_v, sid)
    _combine_partials(si_t, tmp_v, sip_hbm)
    _combine_partials(sj_t, tmp_v, sjp_hbm)
    pltpu.sync_copy(ev_hbm.at[wid], ev_v)
    pltpu.sync_copy(i_hbm.at[wid], i_v)
    pltpu.sync_copy(j_hbm.at[wid], j_v)

    @plsc.parallel_loop(0, NCH, unroll=2)
    def _(b):
        for k in range(CH // L):
            col = pl.ds(k * L, L)
            s = pl.ds(b * CH + k * L, L)
            i16 = i_v[b, col]
            j16 = j_v[b, col]
            ev16 = ev_v[s]
            gi = plsc.load_gather(si_t, [i16])
            gj = plsc.load_gather(sj_t, [j16])
            ev_v[s] = jnp.exp(ev16 / gi + ev16 / gj)

    plsc.subcore_barrier()

    @pl.loop(0, NCH, step=5)
    def _(b):
        descs = []
        for t in range(5):
            descs.append(pltpu.async_copy(
                ev_v.at[pl.ds((b + t) * CH, CH)],
                se_sh.at[j_v.at[b + t]], sem_e, add=True))
        for d in descs:
            d.wait()

    pltpu.sync_copy(ev_v, w_hbm.at[wid])
    plsc.subcore_barrier()
    _export_table(se_sh, sep_hbm, cid, sid)


@functools.partial(
    pl.kernel,
    out_type=[
        jax.ShapeDtypeStruct((NW, EPW), _f32),
    ],
    mesh=_mesh,
    compiler_params=_sc_params,
    scratch_types=[
        pltpu.VMEM((EPW,), _f32),
        pltpu.VMEM((NCH, CH), jnp.int32),
        pltpu.VMEM((NP,), _f32),
        pltpu.VMEM((NP,), _f32),
    ],
)
def _stage_b2(w_hbm, j_hbm, sep_hbm, a_hbm, w_v, j_v, se_t, tmp_v):
    cid, sid, wid = _worker_ids()
    _combine_partials(se_t, tmp_v, sep_hbm)
    pltpu.sync_copy(w_hbm.at[wid], w_v)
    pltpu.sync_copy(j_hbm.at[wid], j_v)

    @plsc.parallel_loop(0, NCH, unroll=2)
    def _(b):
        for k in range(CH // L):
            s = pl.ds(b * CH + k * L, L)
            j16 = j_v[b, pl.ds(k * L, L)]
            w_v[s] = w_v[s] / plsc.load_gather(se_t, [j16])

    pltpu.sync_copy(w_v, a_hbm.at[wid])


@functools.partial(
    pl.kernel,
    out_type=[
        jax.ShapeDtypeStruct((NC, NP, D), _f32),
    ],
    mesh=_mesh,
    compiler_params=_sc_params,
    scratch_types=[
        pltpu.VMEM((SUPC * CC,), _f32),
        pltpu.VMEM((SUPC, CC), jnp.int32),
        pltpu.VMEM((SUPC, CC), jnp.int32),
        pltpu.VMEM((8, D), _f32),
    ] + [pltpu.VMEM((CC, D), _f32)] * NBUF
      + [pltpu.VMEM_SHARED((NP, D), _f32)]
      + [pltpu.SemaphoreType.DMA] * (2 * NBUF),
)
def _stage_c(x_hbm, a_hbm, i_hbm, j_hbm, p_hbm,
             a_v, i_v, j_v, zrows_v, *rest):
    xbs = rest[:NBUF]
    out_sh = rest[NBUF]
    gsems = rest[NBUF + 1:2 * NBUF + 1]
    ssems = rest[2 * NBUF + 1:]
    cid, sid, wid = _worker_ids()

    @pl.loop(0, 8)
    def _(t):
        for q in range(D // L):
            zrows_v[t, pl.ds(q * L, L)] = jnp.zeros((L,), _f32)

    @pl.loop(0, SUBN // 8)
    def _(t):
        pltpu.sync_copy(zrows_v, out_sh.at[pl.ds(sid * SUBN + t * 8, 8)])

    plsc.subcore_barrier()

    @pl.loop(0, NSUP)
    def _(sc):
        pltpu.sync_copy(a_hbm.at[wid, sc], a_v)
        pltpu.sync_copy(i_hbm.at[wid, sc], i_v)
        pltpu.sync_copy(j_hbm.at[wid, sc], j_v)

        gd = {}
        sd = {}
        for b in range(NBUF - 1):
            gd[b] = pltpu.async_copy(x_hbm.at[j_v.at[b]], xbs[b], gsems[b])
        for b in range(SUPC):
            bb = b % NBUF
            gd[b].wait()
            xb = xbs[bb]

            @plsc.parallel_loop(0, CC, unroll=4)
            def _(r):
                av = plsc.load_gather(
                    a_v, [jnp.full((L,), b * CC + r, jnp.int32)])
                for q in range(D // L):
                    col = pl.ds(q * L, L)
                    xb[r, col] = xb[r, col] * av

            sd[b] = pltpu.async_copy(xb, out_sh.at[i_v.at[b]], ssems[bb],
                                     add=True)
            if b + NBUF - 1 < SUPC:
                nb = (b + NBUF - 1) % NBUF
                if b - 1 >= 0:
                    sd[b - 1].wait()
                gd[b + NBUF - 1] = pltpu.async_copy(
                    x_hbm.at[j_v.at[b + NBUF - 1]], xbs[nb], gsems[nb])
        for b in range(max(0, SUPC - NBUF), SUPC):
            sd[b].wait()

    plsc.subcore_barrier()
    rows = pl.ds(sid * SUBN, SUBN)
    pltpu.sync_copy(out_sh.at[rows], p_hbm.at[cid, rows])


def _finish_body(p0_ref, p1_ref, o_ref):
    o_ref[...] = jnp.maximum(p0_ref[0] + p1_ref[0], 0.0)


_finish = pl.pallas_call(
    _finish_body,
    out_shape=jax.ShapeDtypeStruct((N, D), _f32),
    grid=(5,),
    in_specs=[
        pl.BlockSpec((1, 2000, D), lambda i: (0, i, 0)),
        pl.BlockSpec((1, 2000, D), lambda i: (1, i, 0)),
    ],
    out_specs=pl.BlockSpec((2000, D), lambda i: (i, 0)),
)


def kernel(x, edge_index, val):
    ei = edge_index.astype(jnp.int32)
    j3 = ei[0].reshape(NW, NCH, CH)
    i3 = ei[1].reshape(NW, NCH, CH)
    val2 = val.astype(_f32).reshape(NW, EPW)
    ev, sip, sjp = _stage_a(val2, i3, j3)
    w, sep = _stage_b(ev, i3, j3, sip, sjp)
    (alpha,) = _stage_b2(w, j3, sep)
    j4 = ei[0].reshape(NW, NSUP, SUPC, CC)
    i4 = ei[1].reshape(NW, NSUP, SUPC, CC)
    a4 = alpha.reshape(NW, NSUP, SUPC * CC)
    (p,) = _stage_c(x.astype(_f32), a4, i4, j4)
    return _finish(p, p)


# speedup vs baseline: 1.1332x; 1.1332x over previous
import dataclasses
import functools

import jax
import jax.numpy as jnp
from jax import lax
from jax.experimental import pallas as pl
from jax.experimental.pallas import tpu as pltpu
from jax.experimental.pallas import tpu_sc as plsc

N = 10000
E = 320000
D = 128
NC = 2
NS = 16
NW = NC * NS
EPW = E // NW
CH = 80
NCH = EPW // CH
CC = 40
SUPC = 25
NSUP = EPW // (SUPC * CC)
NBUF = 5
NP = 10240
SUBN = NP // NS
L = 16

_mesh = plsc.VectorSubcoreMesh(
    core_axis_name="c", subcore_axis_name="s", num_cores=NC, num_subcores=NS
)

_f32 = jnp.float32

_sc_params = pltpu.CompilerParams()
if "needs_layout_passes" in pltpu.CompilerParams.__dataclass_fields__:
    _sc_params = dataclasses.replace(_sc_params, needs_layout_passes=False)


def _worker_ids():
    cid = lax.axis_index("c")
    sid = lax.axis_index("s")
    return cid, sid, cid * NS + sid


def _zero_vec(vec_ref):
    @pl.loop(0, vec_ref.shape[0] // L)
    def _(t):
        vec_ref[pl.ds(t * L, L)] = jnp.zeros((L,), _f32)


def _zero_shared_table(sh_ref, zero_ref, sid):
    pltpu.sync_copy(zero_ref, sh_ref.at[pl.ds(sid * SUBN, SUBN)])


def _combine_partials(dst_ref, tmp_ref, part_hbm):
    pltpu.sync_copy(part_hbm.at[0], dst_ref)
    pltpu.sync_copy(part_hbm.at[1], tmp_ref)

    @plsc.parallel_loop(0, NP // L, unroll=4)
    def _(t):
        s = pl.ds(t * L, L)
        dst_ref[s] = dst_ref[s] + tmp_ref[s]


def _export_table(sh_ref, part_hbm, cid, sid):
    s = pl.ds(sid * SUBN, SUBN)
    pltpu.sync_copy(sh_ref.at[s], part_hbm.at[cid, s])


@functools.partial(
    pl.kernel,
    out_type=[
        jax.ShapeDtypeStruct((NW, EPW), _f32),
        jax.ShapeDtypeStruct((NC, NP), _f32),
        jax.ShapeDtypeStruct((NC, NP), _f32),
    ],
    mesh=_mesh,
    compiler_params=_sc_params,
    scratch_types=[
        pltpu.VMEM((EPW,), _f32),
        pltpu.VMEM((NCH, CH), jnp.int32),
        pltpu.VMEM((NCH, CH), jnp.int32),
        pltpu.VMEM((SUBN,), _f32),
        pltpu.VMEM_SHARED((NP,), _f32),
        pltpu.VMEM_SHARED((NP,), _f32),
        pltpu.SemaphoreType.DMA,
        pltpu.SemaphoreType.DMA,
    ],
)
def _stage_a(val_hbm, i_hbm, j_hbm, ev_hbm, sip_hbm, sjp_hbm,
             val_v, i_v, j_v, zero_v, si_sh, sj_sh, sem_i, sem_j):
    cid, sid, wid = _worker_ids()
    _zero_vec(zero_v)
    _zero_shared_table(si_sh, zero_v, sid)
    _zero_shared_table(sj_sh, zero_v, sid)
    pltpu.sync_copy(val_hbm.at[wid], val_v)
    pltpu.sync_copy(i_hbm.at[wid], i_v)
    pltpu.sync_copy(j_hbm.at[wid], j_v)

    @plsc.parallel_loop(0, EPW // L, unroll=4)
    def _(t):
        s = pl.ds(t * L, L)
        val_v[s] = jnp.exp(val_v[s])

    plsc.subcore_barrier()

    @pl.loop(0, NCH, step=5)
    def _(b):
        descs = []
        for t in range(5):
            src = val_v.at[pl.ds((b + t) * CH, CH)]
            descs.append(pltpu.async_copy(src, si_sh.at[i_v.at[b + t]],
                                          sem_i, add=True))
            descs.append(pltpu.async_copy(src, sj_sh.at[j_v.at[b + t]],
                                          sem_j, add=True))
        for d in descs:
            d.wait()

    pltpu.sync_copy(val_v, ev_hbm.at[wid])
    plsc.subcore_barrier()
    _export_table(si_sh, sip_hbm, cid, sid)
    _export_table(sj_sh, sjp_hbm, cid, sid)


@functools.partial(
    pl.kernel,
    out_type=[
        jax.ShapeDtypeStruct((NW, EPW), _f32),
        jax.ShapeDtypeStruct((NC, NP), _f32),
    ],
    mesh=_mesh,
    compiler_params=_sc_params,
    scratch_types=[
        pltpu.VMEM((EPW,), _f32),
        pltpu.VMEM((NCH, CH), jnp.int32),
        pltpu.VMEM((NCH, CH), jnp.int32),
        pltpu.VMEM((NP,), _f32),
        pltpu.VMEM((NP,), _f32),
        pltpu.VMEM((NP,), _f32),
        pltpu.VMEM((SUBN,), _f32),
        pltpu.VMEM_SHARED((NP,), _f32),
        pltpu.SemaphoreType.DMA,
    ],
)
def _stage_b(ev_hbm, i_hbm, j_hbm, sip_hbm, sjp_hbm, w_hbm, sep_hbm,
             ev_v, i_v, j_v, si_t, sj_t, tmp_v, zero_v, se_sh, sem_e):
    cid, sid, wid = _worker_ids()
    _zero_vec(zero_v)
    _zero_shared_table(se_sh, zero_v, sid)
    _combine_partials(si_t, tmp_v, sip_hbm)
    _combine_partials(sj_t, tmp_v, sjp_hbm)
    pltpu.sync_copy(ev_hbm.at[wid], ev_v)
    pltpu.sync_copy(i_hbm.at[wid], i_v)
    pltpu.sync_copy(j_hbm.at[wid], j_v)

    @plsc.parallel_loop(0, NCH, unroll=2)
    def _(b):
        for k in range(CH // L):
            col = pl.ds(k * L, L)
            s = pl.ds(b * CH + k * L, L)
            i16 = i_v[b, col]
            j16 = j_v[b, col]
            ev16 = ev_v[s]
            gi = plsc.load_gather(si_t, [i16])
            gj = plsc.load_gather(sj_t, [j16])
            ev_v[s] = jnp.exp(ev16 / gi + ev16 / gj)

    plsc.subcore_barrier()

    @pl.loop(0, NCH, step=5)
    def _(b):
        descs = []
        for t in range(5):
            descs.append(pltpu.async_copy(
                ev_v.at[pl.ds((b + t) * CH, CH)],
                se_sh.at[j_v.at[b + t]], sem_e, add=True))
        for d in descs:
            d.wait()

    pltpu.sync_copy(ev_v, w_hbm.at[wid])
    plsc.subcore_barrier()
    _export_table(se_sh, sep_hbm, cid, sid)


@functools.partial(
    pl.kernel,
    out_type=[
        jax.ShapeDtypeStruct((NW, EPW), _f32),
    ],
    mesh=_mesh,
    compiler_params=_sc_params,
    scratch_types=[
        pltpu.VMEM((EPW,), _f32),
        pltpu.VMEM((NCH, CH), jnp.int32),
        pltpu.VMEM((NP,), _f32),
        pltpu.VMEM((NP,), _f32),
    ],
)
def _stage_b2(w_hbm, j_hbm, sep_hbm, a_hbm, w_v, j_v, se_t, tmp_v):
    cid, sid, wid = _worker_ids()
    _combine_partials(se_t, tmp_v, sep_hbm)
    pltpu.sync_copy(w_hbm.at[wid], w_v)
    pltpu.sync_copy(j_hbm.at[wid], j_v)

    @plsc.parallel_loop(0, NCH, unroll=2)
    def _(b):
        for k in range(CH // L):
            s = pl.ds(b * CH + k * L, L)
            j16 = j_v[b, pl.ds(k * L, L)]
            w_v[s] = w_v[s] / plsc.load_gather(se_t, [j16])

    pltpu.sync_copy(w_v, a_hbm.at[wid])


@functools.partial(
    pl.kernel,
    out_type=[
        jax.ShapeDtypeStruct((NC, NP, D), _f32),
    ],
    mesh=_mesh,
    compiler_params=_sc_params,
    scratch_types=[
        pltpu.VMEM((SUPC * CC,), _f32),
        pltpu.VMEM((SUPC, CC), jnp.int32),
        pltpu.VMEM((SUPC, CC), jnp.int32),
        pltpu.VMEM((8, D), _f32),
    ] + [pltpu.VMEM((CC, D), _f32)] * NBUF
      + [pltpu.VMEM_SHARED((NP, D), _f32)]
      + [pltpu.SemaphoreType.DMA] * (2 * NBUF),
)
def _stage_c(x_hbm, a_hbm, i_hbm, j_hbm, p_hbm,
             a_v, i_v, j_v, zrows_v, *rest):
    xbs = rest[:NBUF]
    out_sh = rest[NBUF]
    gsems = rest[NBUF + 1:2 * NBUF + 1]
    ssems = rest[2 * NBUF + 1:]
    cid, sid, wid = _worker_ids()

    @pl.loop(0, 8)
    def _(t):
        for q in range(D // L):
            zrows_v[t, pl.ds(q * L, L)] = jnp.zeros((L,), _f32)

    @pl.loop(0, SUBN // 8)
    def _(t):
        pltpu.sync_copy(zrows_v, out_sh.at[pl.ds(sid * SUBN + t * 8, 8)])

    plsc.subcore_barrier()

    @pl.loop(0, NSUP)
    def _(sc):
        pltpu.sync_copy(a_hbm.at[wid, sc], a_v)
        pltpu.sync_copy(i_hbm.at[wid, sc], i_v)
        pltpu.sync_copy(j_hbm.at[wid, sc], j_v)

        gd = {}
        sd = {}
        for b in range(NBUF - 1):
            gd[b] = pltpu.async_copy(x_hbm.at[j_v.at[b]], xbs[b], gsems[b])
        for b in range(SUPC):
            bb = b % NBUF
            gd[b].wait()
            xb = xbs[bb]

            @plsc.parallel_loop(0, CC, unroll=4)
            def _(r):
                av = plsc.load_gather(
                    a_v, [jnp.full((L,), b * CC + r, jnp.int32)])
                for q in range(D // L):
                    col = pl.ds(q * L, L)
                    xb[r, col] = xb[r, col] * av

            sd[b] = pltpu.async_copy(xb, out_sh.at[i_v.at[b]], ssems[bb],
                                     add=True)
            if b + NBUF - 1 < SUPC:
                nb = (b + NBUF - 1) % NBUF
                if b - 1 >= 0:
                    sd[b - 1].wait()
                gd[b + NBUF - 1] = pltpu.async_copy(
                    x_hbm.at[j_v.at[b + NBUF - 1]], xbs[nb], gsems[nb])
        for b in range(max(0, SUPC - NBUF), SUPC):
            sd[b].wait()

    plsc.subcore_barrier()
    rows = pl.ds(sid * SUBN, SUBN)
    pltpu.sync_copy(out_sh.at[rows], p_hbm.at[cid, rows])


def _finish_body(p0_ref, p1_ref, o_ref):
    o_ref[...] = jnp.maximum(p0_ref[0] + p1_ref[0], 0.0)


_finish = pl.pallas_call(
    _finish_body,
    out_shape=jax.ShapeDtypeStruct((N, D), _f32),
    grid=(5,),
    in_specs=[
        pl.BlockSpec((1, 2000, D), lambda i: (0, i, 0)),
        pl.BlockSpec((1, 2000, D), lambda i: (1, i, 0)),
    ],
    out_specs=pl.BlockSpec((2000, D), lambda i: (i, 0)),
)


def kernel(x, edge_index, val):
    ei = edge_index.astype(jnp.int32)
    j3 = ei[0].reshape(NW, NCH, CH)
    i3 = ei[1].reshape(NW, NCH, CH)
    val2 = val.astype(_f32).reshape(NW, EPW)
    ev, sip, sjp = _stage_a(val2, i3, j3)
    w, sep = _stage_b(ev, i3, j3, sip, sjp)
    (alpha,) = _stage_b2(w, j3, sep)
    j4 = ei[0].reshape(NW, NSUP, SUPC, CC)
    i4 = ei[1].reshape(NW, NSUP, SUPC, CC)
    a4 = alpha.reshape(NW, NSUP, SUPC * CC)
    (p,) = _stage_c(x.astype(_f32), a4, i4, j4)
    return _finish(p, p)


# speedup vs baseline: 1.1888x; 1.0491x over previous
import dataclasses
import functools

import jax
import jax.numpy as jnp
from jax import lax
from jax.experimental import pallas as pl
from jax.experimental.pallas import tpu as pltpu
from jax.experimental.pallas import tpu_sc as plsc

N = 10000
E = 320000
D = 128
NC = 2
NS = 16
NW = NC * NS
EPW = E // NW
CH = 80
NCH = EPW // CH
CC = 40
SUPC = 25
NSUP = EPW // (SUPC * CC)
NBUF = 5
NP = 10240
SUBN = NP // NS
L = 16

_mesh = plsc.VectorSubcoreMesh(
    core_axis_name="c", subcore_axis_name="s", num_cores=NC, num_subcores=NS
)

_f32 = jnp.float32

_sc_params = pltpu.CompilerParams()
if "needs_layout_passes" in pltpu.CompilerParams.__dataclass_fields__:
    _sc_params = dataclasses.replace(_sc_params, needs_layout_passes=False)


def _worker_ids():
    cid = lax.axis_index("c")
    sid = lax.axis_index("s")
    return cid, sid, cid * NS + sid


def _zero_vec(vec_ref):
    @pl.loop(0, vec_ref.shape[0] // L)
    def _(t):
        vec_ref[pl.ds(t * L, L)] = jnp.zeros((L,), _f32)


def _zero_shared_table(sh_ref, zero_ref, sid):
    pltpu.sync_copy(zero_ref, sh_ref.at[pl.ds(sid * SUBN, SUBN)])


def _combine_partials(dst_ref, tmp_ref, part_hbm):
    pltpu.sync_copy(part_hbm.at[0], dst_ref)
    pltpu.sync_copy(part_hbm.at[1], tmp_ref)

    @plsc.parallel_loop(0, NP // L, unroll=4)
    def _(t):
        s = pl.ds(t * L, L)
        dst_ref[s] = dst_ref[s] + tmp_ref[s]


def _export_table(sh_ref, part_hbm, cid, sid):
    s = pl.ds(sid * SUBN, SUBN)
    pltpu.sync_copy(sh_ref.at[s], part_hbm.at[cid, s])


@functools.partial(
    pl.kernel,
    out_type=[
        jax.ShapeDtypeStruct((NW, EPW), _f32),
        jax.ShapeDtypeStruct((NC, NP), _f32),
        jax.ShapeDtypeStruct((NC, NP), _f32),
    ],
    mesh=_mesh,
    compiler_params=_sc_params,
    scratch_types=[
        pltpu.VMEM((EPW,), _f32),
        pltpu.VMEM((NCH, CH), jnp.int32),
        pltpu.VMEM((NCH, CH), jnp.int32),
        pltpu.VMEM((SUBN,), _f32),
        pltpu.VMEM_SHARED((NP,), _f32),
        pltpu.VMEM_SHARED((NP,), _f32),
        pltpu.SemaphoreType.DMA,
        pltpu.SemaphoreType.DMA,
    ],
)
def _stage_a(val_hbm, i_hbm, j_hbm, ev_hbm, sip_hbm, sjp_hbm,
             val_v, i_v, j_v, zero_v, si_sh, sj_sh, sem_i, sem_j):
    cid, sid, wid = _worker_ids()
    _zero_vec(zero_v)
    _zero_shared_table(si_sh, zero_v, sid)
    _zero_shared_table(sj_sh, zero_v, sid)
    pltpu.sync_copy(val_hbm.at[wid], val_v)
    pltpu.sync_copy(i_hbm.at[wid], i_v)
    pltpu.sync_copy(j_hbm.at[wid], j_v)

    @plsc.parallel_loop(0, EPW // L, unroll=4)
    def _(t):
        s = pl.ds(t * L, L)
        val_v[s] = jnp.exp(val_v[s])

    plsc.subcore_barrier()

    @pl.loop(0, NCH, step=5)
    def _(b):
        descs = []
        for t in range(5):
            src = val_v.at[pl.ds((b + t) * CH, CH)]
            descs.append(pltpu.async_copy(src, si_sh.at[i_v.at[b + t]],
                                          sem_i, add=True))
            descs.append(pltpu.async_copy(src, sj_sh.at[j_v.at[b + t]],
                                          sem_j, add=True))
        for d in descs:
            d.wait()

    pltpu.sync_copy(val_v, ev_hbm.at[wid])
    plsc.subcore_barrier()
    _export_table(si_sh, sip_hbm, cid, sid)
    _export_table(sj_sh, sjp_hbm, cid, sid)


@functools.partial(
    pl.kernel,
    out_type=[
        jax.ShapeDtypeStruct((NW, EPW), _f32),
        jax.ShapeDtypeStruct((NC, NP), _f32),
    ],
    mesh=_mesh,
    compiler_params=_sc_params,
    scratch_types=[
        pltpu.VMEM((EPW,), _f32),
        pltpu.VMEM((NCH, CH), jnp.int32),
        pltpu.VMEM((NCH, CH), jnp.int32),
        pltpu.VMEM((NP,), _f32),
        pltpu.VMEM((NP,), _f32),
        pltpu.VMEM((NP,), _f32),
        pltpu.VMEM((SUBN,), _f32),
        pltpu.VMEM_SHARED((NP,), _f32),
        pltpu.SemaphoreType.DMA,
    ],
)
def _stage_b(ev_hbm, i_hbm, j_hbm, sip_hbm, sjp_hbm, w_hbm, sep_hbm,
             ev_v, i_v, j_v, si_t, sj_t, tmp_v, zero_v, se_sh, sem_e):
    cid, sid, wid = _worker_ids()
    _zero_vec(zero_v)
    _zero_shared_table(se_sh, zero_v, sid)
    _combine_partials(si_t, tmp_v, sip_hbm)
    _combine_partials(sj_t, tmp_v, sjp_hbm)
    pltpu.sync_copy(ev_hbm.at[wid], ev_v)
    pltpu.sync_copy(i_hbm.at[wid], i_v)
    pltpu.sync_copy(j_hbm.at[wid], j_v)

    @plsc.parallel_loop(0, NCH, unroll=2)
    def _(b):
        for k in range(CH // L):
            col = pl.ds(k * L, L)
            s = pl.ds(b * CH + k * L, L)
            i16 = i_v[b, col]
            j16 = j_v[b, col]
            ev16 = ev_v[s]
            gi = plsc.load_gather(si_t, [i16])
            gj = plsc.load_gather(sj_t, [j16])
            ev_v[s] = jnp.exp(ev16 / gi + ev16 / gj)

    plsc.subcore_barrier()

    @pl.loop(0, NCH, step=5)
    def _(b):
        descs = []
        for t in range(5):
            descs.append(pltpu.async_copy(
                ev_v.at[pl.ds((b + t) * CH, CH)],
                se_sh.at[j_v.at[b + t]], sem_e, add=True))
        for d in descs:
            d.wait()

    pltpu.sync_copy(ev_v, w_hbm.at[wid])
    plsc.subcore_barrier()
    _export_table(se_sh, sep_hbm, cid, sid)


@functools.partial(
    pl.kernel,
    out_type=[
        jax.ShapeDtypeStruct((NW, EPW), _f32),
    ],
    mesh=_mesh,
    compiler_params=_sc_params,
    scratch_types=[
        pltpu.VMEM((EPW,), _f32),
        pltpu.VMEM((NCH, CH), jnp.int32),
        pltpu.VMEM((NP,), _f32),
        pltpu.VMEM((NP,), _f32),
    ],
)
def _stage_b2(w_hbm, j_hbm, sep_hbm, a_hbm, w_v, j_v, se_t, tmp_v):
    cid, sid, wid = _worker_ids()
    _combine_partials(se_t, tmp_v, sep_hbm)
    pltpu.sync_copy(w_hbm.at[wid], w_v)
    pltpu.sync_copy(j_hbm.at[wid], j_v)

    @plsc.parallel_loop(0, NCH, unroll=2)
    def _(b):
        for k in range(CH // L):
            s = pl.ds(b * CH + k * L, L)
            j16 = j_v[b, pl.ds(k * L, L)]
            w_v[s] = w_v[s] / plsc.load_gather(se_t, [j16])

    pltpu.sync_copy(w_v, a_hbm.at[wid])


@functools.partial(
    pl.kernel,
    out_type=[
        jax.ShapeDtypeStruct((NC, NP, D), _f32),
    ],
    mesh=_mesh,
    compiler_params=_sc_params,
    scratch_types=[
        pltpu.VMEM((SUPC * CC,), _f32),
        pltpu.VMEM((SUPC, CC), jnp.int32),
        pltpu.VMEM((SUPC, CC), jnp.int32),
        pltpu.VMEM((8, D), _f32),
    ] + [pltpu.VMEM((CC, D), _f32)] * NBUF
      + [pltpu.VMEM_SHARED((NP, D), _f32)]
      + [pltpu.SemaphoreType.DMA] * (2 * NBUF),
)
def _stage_c(x_hbm, a_hbm, i_hbm, j_hbm, p_hbm,
             a_v, i_v, j_v, zrows_v, *rest):
    xbs = rest[:NBUF]
    out_sh = rest[NBUF]
    gsems = rest[NBUF + 1:2 * NBUF + 1]
    ssems = rest[2 * NBUF + 1:]
    cid, sid, wid = _worker_ids()

    @pl.loop(0, 8)
    def _(t):
        for q in range(D // L):
            zrows_v[t, pl.ds(q * L, L)] = jnp.zeros((L,), _f32)

    @pl.loop(0, SUBN // 8)
    def _(t):
        pltpu.sync_copy(zrows_v, out_sh.at[pl.ds(sid * SUBN + t * 8, 8)])

    plsc.subcore_barrier()

    @pl.loop(0, NSUP)
    def _(sc):
        ld = [pltpu.async_copy(a_hbm.at[wid, sc], a_v, gsems[0]),
              pltpu.async_copy(i_hbm.at[wid, sc], i_v, gsems[1]),
              pltpu.async_copy(j_hbm.at[wid, sc], j_v, gsems[2])]
        for d in ld:
            d.wait()

        gd = {}
        sd = {}
        for b in range(NBUF - 1):
            gd[b] = pltpu.async_copy(x_hbm.at[j_v.at[b]], xbs[b], gsems[b])
        for b in range(SUPC):
            bb = b % NBUF
            gd[b].wait()
            xb = xbs[bb]

            @plsc.parallel_loop(0, CC, unroll=4)
            def _(r):
                av = plsc.load_gather(
                    a_v, [jnp.full((L,), b * CC + r, jnp.int32)])
                for q in range(D // L):
                    col = pl.ds(q * L, L)
                    xb[r, col] = xb[r, col] * av

            sd[b] = pltpu.async_copy(xb, out_sh.at[i_v.at[b]], ssems[bb],
                                     add=True)
            if b + NBUF - 1 < SUPC:
                nb = (b + NBUF - 1) % NBUF
                if b - 1 >= 0:
                    sd[b - 1].wait()
                gd[b + NBUF - 1] = pltpu.async_copy(
                    x_hbm.at[j_v.at[b + NBUF - 1]], xbs[nb], gsems[nb])
        for b in range(max(0, SUPC - NBUF), SUPC):
            sd[b].wait()

    plsc.subcore_barrier()
    rows = pl.ds(sid * SUBN, SUBN)
    pltpu.sync_copy(out_sh.at[rows], p_hbm.at[cid, rows])


def _finish_body(p0_ref, p1_ref, o_ref):
    o_ref[...] = jnp.maximum(p0_ref[0] + p1_ref[0], 0.0)


_finish = pl.pallas_call(
    _finish_body,
    out_shape=jax.ShapeDtypeStruct((N, D), _f32),
    grid=(5,),
    in_specs=[
        pl.BlockSpec((1, 2000, D), lambda i: (0, i, 0)),
        pl.BlockSpec((1, 2000, D), lambda i: (1, i, 0)),
    ],
    out_specs=pl.BlockSpec((2000, D), lambda i: (i, 0)),
)


def kernel(x, edge_index, val):
    ei = edge_index.astype(jnp.int32)
    j3 = ei[0].reshape(NW, NCH, CH)
    i3 = ei[1].reshape(NW, NCH, CH)
    val2 = val.astype(_f32).reshape(NW, EPW)
    ev, sip, sjp = _stage_a(val2, i3, j3)
    w, sep = _stage_b(ev, i3, j3, sip, sjp)
    (alpha,) = _stage_b2(w, j3, sep)
    j4 = ei[0].reshape(NW, NSUP, SUPC, CC)
    i4 = ei[1].reshape(NW, NSUP, SUPC, CC)
    a4 = alpha.reshape(NW, NSUP, SUPC * CC)
    (p,) = _stage_c(x.astype(_f32), a4, i4, j4)
    return _finish(p, p)


# speedup vs baseline: 1.2035x; 1.0124x over previous
import dataclasses
import functools

import jax
import jax.numpy as jnp
from jax import lax
from jax.experimental import pallas as pl
from jax.experimental.pallas import tpu as pltpu
from jax.experimental.pallas import tpu_sc as plsc

N = 10000
E = 320000
D = 128
NC = 2
NS = 16
NW = NC * NS
EPW = E // NW
CH = 80
NCH = EPW // CH
CC = 40
SUPC = 50
NSUP = EPW // (SUPC * CC)
NBUF = 5
NP = 10240
SUBN = NP // NS
L = 16

_mesh = plsc.VectorSubcoreMesh(
    core_axis_name="c", subcore_axis_name="s", num_cores=NC, num_subcores=NS
)

_f32 = jnp.float32

_sc_params = pltpu.CompilerParams()
if "needs_layout_passes" in pltpu.CompilerParams.__dataclass_fields__:
    _sc_params = dataclasses.replace(_sc_params, needs_layout_passes=False)


def _worker_ids():
    cid = lax.axis_index("c")
    sid = lax.axis_index("s")
    return cid, sid, cid * NS + sid


def _zero_vec(vec_ref):
    @pl.loop(0, vec_ref.shape[0] // L)
    def _(t):
        vec_ref[pl.ds(t * L, L)] = jnp.zeros((L,), _f32)


def _zero_shared_table(sh_ref, zero_ref, sid):
    pltpu.sync_copy(zero_ref, sh_ref.at[pl.ds(sid * SUBN, SUBN)])


def _combine_partials(dst_ref, tmp_ref, part_hbm):
    pltpu.sync_copy(part_hbm.at[0], dst_ref)
    pltpu.sync_copy(part_hbm.at[1], tmp_ref)

    @plsc.parallel_loop(0, NP // L, unroll=4)
    def _(t):
        s = pl.ds(t * L, L)
        dst_ref[s] = dst_ref[s] + tmp_ref[s]


def _export_table(sh_ref, part_hbm, cid, sid):
    s = pl.ds(sid * SUBN, SUBN)
    pltpu.sync_copy(sh_ref.at[s], part_hbm.at[cid, s])


@functools.partial(
    pl.kernel,
    out_type=[
        jax.ShapeDtypeStruct((NW, EPW), _f32),
        jax.ShapeDtypeStruct((NC, NP), _f32),
        jax.ShapeDtypeStruct((NC, NP), _f32),
    ],
    mesh=_mesh,
    compiler_params=_sc_params,
    scratch_types=[
        pltpu.VMEM((EPW,), _f32),
        pltpu.VMEM((NCH, CH), jnp.int32),
        pltpu.VMEM((NCH, CH), jnp.int32),
        pltpu.VMEM((SUBN,), _f32),
        pltpu.VMEM_SHARED((NP,), _f32),
        pltpu.VMEM_SHARED((NP,), _f32),
        pltpu.SemaphoreType.DMA,
        pltpu.SemaphoreType.DMA,
    ],
)
def _stage_a(val_hbm, i_hbm, j_hbm, ev_hbm, sip_hbm, sjp_hbm,
             val_v, i_v, j_v, zero_v, si_sh, sj_sh, sem_i, sem_j):
    cid, sid, wid = _worker_ids()
    _zero_vec(zero_v)
    _zero_shared_table(si_sh, zero_v, sid)
    _zero_shared_table(sj_sh, zero_v, sid)
    pltpu.sync_copy(val_hbm.at[wid], val_v)
    pltpu.sync_copy(i_hbm.at[wid], i_v)
    pltpu.sync_copy(j_hbm.at[wid], j_v)

    @plsc.parallel_loop(0, EPW // L, unroll=4)
    def _(t):
        s = pl.ds(t * L, L)
        val_v[s] = jnp.exp(val_v[s])

    plsc.subcore_barrier()

    @pl.loop(0, NCH, step=5)
    def _(b):
        descs = []
        for t in range(5):
            src = val_v.at[pl.ds((b + t) * CH, CH)]
            descs.append(pltpu.async_copy(src, si_sh.at[i_v.at[b + t]],
                                          sem_i, add=True))
            descs.append(pltpu.async_copy(src, sj_sh.at[j_v.at[b + t]],
                                          sem_j, add=True))
        for d in descs:
            d.wait()

    pltpu.sync_copy(val_v, ev_hbm.at[wid])
    plsc.subcore_barrier()
    _export_table(si_sh, sip_hbm, cid, sid)
    _export_table(sj_sh, sjp_hbm, cid, sid)


@functools.partial(
    pl.kernel,
    out_type=[
        jax.ShapeDtypeStruct((NW, EPW), _f32),
        jax.ShapeDtypeStruct((NC, NP), _f32),
    ],
    mesh=_mesh,
    compiler_params=_sc_params,
    scratch_types=[
        pltpu.VMEM((EPW,), _f32),
        pltpu.VMEM((NCH, CH), jnp.int32),
        pltpu.VMEM((NCH, CH), jnp.int32),
        pltpu.VMEM((NP,), _f32),
        pltpu.VMEM((NP,), _f32),
        pltpu.VMEM((NP,), _f32),
        pltpu.VMEM((SUBN,), _f32),
        pltpu.VMEM_SHARED((NP,), _f32),
        pltpu.SemaphoreType.DMA,
    ],
)
def _stage_b(ev_hbm, i_hbm, j_hbm, sip_hbm, sjp_hbm, w_hbm, sep_hbm,
             ev_v, i_v, j_v, si_t, sj_t, tmp_v, zero_v, se_sh, sem_e):
    cid, sid, wid = _worker_ids()
    _zero_vec(zero_v)
    _zero_shared_table(se_sh, zero_v, sid)
    _combine_partials(si_t, tmp_v, sip_hbm)
    _combine_partials(sj_t, tmp_v, sjp_hbm)
    pltpu.sync_copy(ev_hbm.at[wid], ev_v)
    pltpu.sync_copy(i_hbm.at[wid], i_v)
    pltpu.sync_copy(j_hbm.at[wid], j_v)

    @plsc.parallel_loop(0, NCH, unroll=2)
    def _(b):
        for k in range(CH // L):
            col = pl.ds(k * L, L)
            s = pl.ds(b * CH + k * L, L)
            i16 = i_v[b, col]
            j16 = j_v[b, col]
            ev16 = ev_v[s]
            gi = plsc.load_gather(si_t, [i16])
            gj = plsc.load_gather(sj_t, [j16])
            ev_v[s] = jnp.exp(ev16 / gi + ev16 / gj)

    plsc.subcore_barrier()

    @pl.loop(0, NCH, step=5)
    def _(b):
        descs = []
        for t in range(5):
            descs.append(pltpu.async_copy(
                ev_v.at[pl.ds((b + t) * CH, CH)],
                se_sh.at[j_v.at[b + t]], sem_e, add=True))
        for d in descs:
            d.wait()

    pltpu.sync_copy(ev_v, w_hbm.at[wid])
    plsc.subcore_barrier()
    _export_table(se_sh, sep_hbm, cid, sid)


@functools.partial(
    pl.kernel,
    out_type=[
        jax.ShapeDtypeStruct((NW, EPW), _f32),
    ],
    mesh=_mesh,
    compiler_params=_sc_params,
    scratch_types=[
        pltpu.VMEM((EPW,), _f32),
        pltpu.VMEM((NCH, CH), jnp.int32),
        pltpu.VMEM((NP,), _f32),
        pltpu.VMEM((NP,), _f32),
    ],
)
def _stage_b2(w_hbm, j_hbm, sep_hbm, a_hbm, w_v, j_v, se_t, tmp_v):
    cid, sid, wid = _worker_ids()
    _combine_partials(se_t, tmp_v, sep_hbm)
    pltpu.sync_copy(w_hbm.at[wid], w_v)
    pltpu.sync_copy(j_hbm.at[wid], j_v)

    @plsc.parallel_loop(0, NCH, unroll=2)
    def _(b):
        for k in range(CH // L):
            s = pl.ds(b * CH + k * L, L)
            j16 = j_v[b, pl.ds(k * L, L)]
            w_v[s] = w_v[s] / plsc.load_gather(se_t, [j16])

    pltpu.sync_copy(w_v, a_hbm.at[wid])


@functools.partial(
    pl.kernel,
    out_type=[
        jax.ShapeDtypeStruct((NC, NP, D), _f32),
    ],
    mesh=_mesh,
    compiler_params=_sc_params,
    scratch_types=[
        pltpu.VMEM((SUPC * CC,), _f32),
        pltpu.VMEM((SUPC, CC), jnp.int32),
        pltpu.VMEM((SUPC, CC), jnp.int32),
        pltpu.VMEM((8, D), _f32),
    ] + [pltpu.VMEM((CC, D), _f32)] * NBUF
      + [pltpu.VMEM_SHARED((NP, D), _f32)]
      + [pltpu.SemaphoreType.DMA] * (2 * NBUF),
)
def _stage_c(x_hbm, a_hbm, i_hbm, j_hbm, p_hbm,
             a_v, i_v, j_v, zrows_v, *rest):
    xbs = rest[:NBUF]
    out_sh = rest[NBUF]
    gsems = rest[NBUF + 1:2 * NBUF + 1]
    ssems = rest[2 * NBUF + 1:]
    cid, sid, wid = _worker_ids()

    @pl.loop(0, 8)
    def _(t):
        for q in range(D // L):
            zrows_v[t, pl.ds(q * L, L)] = jnp.zeros((L,), _f32)

    @pl.loop(0, SUBN // 8)
    def _(t):
        pltpu.sync_copy(zrows_v, out_sh.at[pl.ds(sid * SUBN + t * 8, 8)])

    plsc.subcore_barrier()

    @pl.loop(0, NSUP)
    def _(sc):
        ld = [pltpu.async_copy(a_hbm.at[wid, sc], a_v, gsems[0]),
              pltpu.async_copy(i_hbm.at[wid, sc], i_v, gsems[1]),
              pltpu.async_copy(j_hbm.at[wid, sc], j_v, gsems[2])]
        for d in ld:
            d.wait()

        gd = {}
        sd = {}
        for b in range(NBUF - 1):
            gd[b] = pltpu.async_copy(x_hbm.at[j_v.at[b]], xbs[b], gsems[b])
        for b in range(SUPC):
            bb = b % NBUF
            gd[b].wait()
            xb = xbs[bb]

            @plsc.parallel_loop(0, CC, unroll=4)
            def _(r):
                av = plsc.load_gather(
                    a_v, [jnp.full((L,), b * CC + r, jnp.int32)])
                for q in range(D // L):
                    col = pl.ds(q * L, L)
                    xb[r, col] = xb[r, col] * av

            sd[b] = pltpu.async_copy(xb, out_sh.at[i_v.at[b]], ssems[bb],
                                     add=True)
            if b + NBUF - 1 < SUPC:
                nb = (b + NBUF - 1) % NBUF
                if b - 1 >= 0:
                    sd[b - 1].wait()
                gd[b + NBUF - 1] = pltpu.async_copy(
                    x_hbm.at[j_v.at[b + NBUF - 1]], xbs[nb], gsems[nb])
        for b in range(max(0, SUPC - NBUF), SUPC):
            sd[b].wait()

    plsc.subcore_barrier()
    rows = pl.ds(sid * SUBN, SUBN)
    pltpu.sync_copy(out_sh.at[rows], p_hbm.at[cid, rows])


def _finish_body(p0_ref, p1_ref, o_ref):
    o_ref[...] = jnp.maximum(p0_ref[0] + p1_ref[0], 0.0)


_finish = pl.pallas_call(
    _finish_body,
    out_shape=jax.ShapeDtypeStruct((N, D), _f32),
    grid=(5,),
    in_specs=[
        pl.BlockSpec((1, 2000, D), lambda i: (0, i, 0)),
        pl.BlockSpec((1, 2000, D), lambda i: (1, i, 0)),
    ],
    out_specs=pl.BlockSpec((2000, D), lambda i: (i, 0)),
)


def kernel(x, edge_index, val):
    ei = edge_index.astype(jnp.int32)
    j3 = ei[0].reshape(NW, NCH, CH)
    i3 = ei[1].reshape(NW, NCH, CH)
    val2 = val.astype(_f32).reshape(NW, EPW)
    ev, sip, sjp = _stage_a(val2, i3, j3)
    w, sep = _stage_b(ev, i3, j3, sip, sjp)
    (alpha,) = _stage_b2(w, j3, sep)
    j4 = ei[0].reshape(NW, NSUP, SUPC, CC)
    i4 = ei[1].reshape(NW, NSUP, SUPC, CC)
    a4 = alpha.reshape(NW, NSUP, SUPC * CC)
    (p,) = _stage_c(x.astype(_f32), a4, i4, j4)
    return _finish(p, p)
